# baseline (speedup 1.0000x reference)
import functools

import jax
import jax.numpy as jnp
from jax import lax
from jax.experimental import pallas as pl
from jax.experimental.pallas import tpu as pltpu
from jax.experimental.pallas import tpu_sc as plsc

_NC, _NS, _L = 2, 16, 16
_NW = _NC * _NS


@functools.cache
def _tc_flatten(ld, v, vp, blkv):
    def body(in_ref, o_ref):
        o_ref[...] = in_ref[...]

    return pl.pallas_call(
        body,
        grid=(vp // blkv,),
        in_specs=[pl.BlockSpec((ld, blkv), lambda i: (0, i))],
        out_specs=pl.BlockSpec((ld, blkv), lambda i: (0, i)),
        out_shape=jax.ShapeDtypeStruct((ld, vp), jnp.float32),
    )


@functools.cache
def _tc_wflatten(v, d, blkv):
    def body(in_ref, o_ref):
        t = in_ref[...].T.astype(jnp.bfloat16)
        o_ref[...] = jnp.concatenate([t, t], axis=1)

    nblk = -(-v // blkv)
    return pl.pallas_call(
        body,
        grid=(nblk,),
        in_specs=[pl.BlockSpec((d, blkv), lambda i: (0, i))],
        out_specs=pl.BlockSpec((blkv, 2 * d), lambda i: (i, 0)),
        out_shape=jax.ShapeDtypeStruct((v, 2 * d), jnp.bfloat16),
    )


@functools.cache
def _sc_gather(b, v, d, ld, vp):
    b_per_w = b // _NW
    e_per_w = b_per_w * ld
    d2 = 2 * d
    mesh = plsc.VectorSubcoreMesh(
        core_axis_name="c", subcore_axis_name="s",
        num_cores=_NC, num_subcores=_NS)

    @functools.partial(
        pl.kernel,
        out_type=[
            jax.ShapeDtypeStruct((b, d2), jnp.bfloat16),
            jax.ShapeDtypeStruct((b * ld,), jnp.float32),
        ],
        mesh=mesh,
        scratch_types=[
            pltpu.VMEM((b_per_w,), jnp.int32),
            pltpu.VMEM((b_per_w,), jnp.int32),
            pltpu.VMEM((b_per_w, d2), jnp.bfloat16),
            pltpu.VMEM((e_per_w,), jnp.int32),
            pltpu.VMEM((e_per_w,), jnp.float32),
            pltpu.SemaphoreType.DMA,
            pltpu.SemaphoreType.DMA,
        ],
        compiler_params=pltpu.CompilerParams(use_tc_tiling_on_sc=False),
    )
    def gather_kernel(w128_hbm, lflat_hbm, idx_hbm, rows_out, a_out,
                      idx_v, idxh_v, rows_v, eidx_v, a_v, sem_w, sem_a):
        wid = lax.axis_index("s") * _NC + lax.axis_index("c")
        base = wid * b_per_w
        pltpu.sync_copy(idx_hbm.at[pl.ds(base, b_per_w)], idx_v)
        cp_w = pltpu.async_copy(w128_hbm.at[idx_v], rows_v, sem_w)

        def jb_body(jb, carry):
            blk = idx_v[pl.ds(jb * _L, _L)]
            for r in range(ld):
                eidx_v[pl.ds(r * b_per_w + jb * _L, _L)] = blk + r * vp
            return carry

        lax.fori_loop(0, b_per_w // _L, jb_body, 0)

        cp_a = pltpu.async_copy(lflat_hbm.at[eidx_v], a_v, sem_a)
        cp_w.wait()
        pltpu.sync_copy(rows_v, rows_out.at[pl.ds(base, b_per_w)])
        cp_a.wait()
        pltpu.sync_copy(a_v, a_out.at[pl.ds(wid * e_per_w, e_per_w)])

    return gather_kernel


@functools.cache
def _tc_epilogue(b, d, ld, b_per_w):
    scale = 1.0 / ld

    def body(rows_ref, a_ref, right_ref, o_ref):
        sel = rows_ref[:, :d].astype(jnp.float32)
        lora = lax.dot_general(
            a_ref[0], right_ref[...],
            (((0,), (1,)), ((), ())),
            preferred_element_type=jnp.float32)
        o_ref[...] = sel + lora * scale

    return pl.pallas_call(
        body,
        grid=(b // b_per_w,),
        in_specs=[
            pl.BlockSpec((b_per_w, 2 * d), lambda i: (i, 0)),
            pl.BlockSpec((1, ld, b_per_w), lambda i: (i, 0, 0)),
            pl.BlockSpec((d, ld), lambda i: (0, 0)),
        ],
        out_specs=pl.BlockSpec((b_per_w, d), lambda i: (i, 0)),
        out_shape=jax.ShapeDtypeStruct((b, d), jnp.float32),
    )


def kernel(input_, weight, lora_left_weight, lora_right_weight):
    b = input_.shape[0]
    v, d = weight.shape
    ld = lora_left_weight.shape[0]
    b_per_w = b // _NW
    vp = 1 << (v - 1).bit_length()
    w128 = _tc_wflatten(v, d, 8192)(weight.T)
    lflat = _tc_flatten(ld, v, vp, vp // 16)(lora_left_weight).reshape(-1)
    rows2, a_flat = _sc_gather(b, v, d, ld, vp)(w128, lflat, input_)
    a_t = a_flat.reshape(_NW, ld, b_per_w)
    return _tc_epilogue(b, d, ld, b_per_w)(rows2, a_t, lora_right_weight)


# speedup vs baseline: 2.8452x; 2.8452x over previous
import functools

import jax
import jax.numpy as jnp
from jax import lax
from jax.experimental import pallas as pl
from jax.experimental.pallas import tpu as pltpu
from jax.experimental.pallas import tpu_sc as plsc

_NC, _NS, _L = 2, 16, 16
_NW = _NC * _NS


@functools.cache
def _tc_flatten(ld, v, vp, blkv):
    def body(in_ref, o_ref):
        o_ref[...] = in_ref[...]

    return pl.pallas_call(
        body,
        grid=(vp // blkv,),
        in_specs=[pl.BlockSpec((ld, blkv), lambda i: (0, i))],
        out_specs=pl.BlockSpec((ld, blkv), lambda i: (0, i)),
        out_shape=jax.ShapeDtypeStruct((ld, vp), jnp.float32),
    )


@functools.cache
def _tc_wflatten(v, d, blkv):
    def body(in_ref, o_ref):
        t = in_ref[...].T
        o_ref[...] = jnp.concatenate([t, t], axis=1)

    nblk = -(-v // blkv)
    return pl.pallas_call(
        body,
        grid=(nblk,),
        in_specs=[pl.BlockSpec((d, blkv), lambda i: (0, i))],
        out_specs=pl.BlockSpec((blkv, 2 * d), lambda i: (i, 0)),
        out_shape=jax.ShapeDtypeStruct((v, 2 * d), jnp.float32),
    )


@functools.cache
def _sc_gather(b, v, d, ld, vp):
    b_per_w = b // _NW
    e_per_w = b_per_w * ld
    d2 = 2 * d
    mesh = plsc.VectorSubcoreMesh(
        core_axis_name="c", subcore_axis_name="s",
        num_cores=_NC, num_subcores=_NS)

    @functools.partial(
        pl.kernel,
        out_type=[
            jax.ShapeDtypeStruct((b, d2), jnp.float32),
            jax.ShapeDtypeStruct((b * ld,), jnp.float32),
        ],
        mesh=mesh,
        scratch_types=[
            pltpu.VMEM((b_per_w,), jnp.int32),
            pltpu.VMEM((b_per_w,), jnp.int32),
            pltpu.VMEM((b_per_w, d2), jnp.float32),
            pltpu.VMEM((e_per_w,), jnp.int32),
            pltpu.VMEM((e_per_w,), jnp.float32),
            pltpu.SemaphoreType.DMA,
            pltpu.SemaphoreType.DMA,
        ],
        compiler_params=pltpu.CompilerParams(use_tc_tiling_on_sc=False),
    )
    def gather_kernel(w128_hbm, lflat_hbm, idx_hbm, rows_out, a_out,
                      idx_v, idxh_v, rows_v, eidx_v, a_v, sem_w, sem_a):
        wid = lax.axis_index("s") * _NC + lax.axis_index("c")
        base = wid * b_per_w
        pltpu.sync_copy(idx_hbm.at[pl.ds(base, b_per_w)], idx_v)
        cp_w = pltpu.async_copy(w128_hbm.at[idx_v], rows_v, sem_w)

        def jb_body(jb, carry):
            blk = idx_v[pl.ds(jb * _L, _L)]
            for r in range(ld):
                eidx_v[pl.ds(r * b_per_w + jb * _L, _L)] = blk + r * vp
            return carry

        lax.fori_loop(0, b_per_w // _L, jb_body, 0)

        cp_a = pltpu.async_copy(lflat_hbm.at[eidx_v], a_v, sem_a)
        cp_w.wait()
        pltpu.sync_copy(rows_v, rows_out.at[pl.ds(base, b_per_w)])
        cp_a.wait()
        pltpu.sync_copy(a_v, a_out.at[pl.ds(wid * e_per_w, e_per_w)])

    return gather_kernel


@functools.cache
def _tc_epilogue(b, d, ld, b_per_w):
    scale = 1.0 / ld

    def body(rows_ref, a_ref, right_ref, o_ref):
        sel = rows_ref[:, :d]
        lora = lax.dot_general(
            a_ref[0], right_ref[...],
            (((0,), (1,)), ((), ())),
            preferred_element_type=jnp.float32)
        o_ref[...] = sel + lora * scale

    return pl.pallas_call(
        body,
        grid=(b // b_per_w,),
        in_specs=[
            pl.BlockSpec((b_per_w, 2 * d), lambda i: (i, 0)),
            pl.BlockSpec((1, ld, b_per_w), lambda i: (i, 0, 0)),
            pl.BlockSpec((d, ld), lambda i: (0, 0)),
        ],
        out_specs=pl.BlockSpec((b_per_w, d), lambda i: (i, 0)),
        out_shape=jax.ShapeDtypeStruct((b, d), jnp.float32),
    )


def kernel(input_, weight, lora_left_weight, lora_right_weight):
    b = input_.shape[0]
    v, d = weight.shape
    ld = lora_left_weight.shape[0]
    b_per_w = b // _NW
    vp = 1 << (v - 1).bit_length()
    w128 = _tc_wflatten(v, d, 16384)(weight.T)
    lflat = _tc_flatten(ld, v, vp, vp // 16)(lora_left_weight).reshape(-1)
    rows2, a_flat = _sc_gather(b, v, d, ld, vp)(w128, lflat, input_)
    a_t = a_flat.reshape(_NW, ld, b_per_w)
    return _tc_epilogue(b, d, ld, b_per_w)(rows2, a_t, lora_right_weight)
